# Initial kernel scaffold; baseline (speedup 1.0000x reference)
import jax
import jax.numpy as jnp
from jax.experimental import pallas as pl


def kernel(user_indices, movie_indices, genre_features, user_emb, movie_emb, user_bias_tab, movie_bias_tab, global_bias, W1, b1, W2, b2):
    raise NotImplementedError("write your pallas kernel here")


# speedup vs baseline: 1.9026x; 1.9026x over previous
import dataclasses
import functools

import jax
import jax.numpy as jnp
from jax import lax
from jax.experimental import pallas as pl
from jax.experimental.pallas import tpu as pltpu
from jax.experimental.pallas import tpu_sc as plsc

BATCH = 16384
EMBED_DIM = 128
NUM_GENRES = 32
HIDDEN_DIM = 1024

NC = 2
NS = 16
NW = NC * NS
BPW = BATCH // NW
CHUNK = 128
NCHUNK = BPW // CHUNK
NLANE = 16


def _sc_gather(user_emb, movie_emb, ubt2, mbt2, uidx2, midx2):
    mesh = plsc.VectorSubcoreMesh(core_axis_name="c", subcore_axis_name="s")
    out_types = (
        jax.ShapeDtypeStruct((BATCH, EMBED_DIM), jnp.float32),
        jax.ShapeDtypeStruct((BATCH, EMBED_DIM), jnp.float32),
        jax.ShapeDtypeStruct((BATCH,), jnp.float32),
        jax.ShapeDtypeStruct((BATCH,), jnp.float32),
    )

    cp = pltpu.CompilerParams()
    if "needs_layout_passes" in pltpu.CompilerParams.__dataclass_fields__:
        cp = dataclasses.replace(cp, needs_layout_passes=False)

    @functools.partial(
        pl.kernel,
        mesh=mesh,
        out_type=out_types,
        compiler_params=cp,
        scratch_types=[
            pltpu.VMEM((NCHUNK, CHUNK), jnp.int32),
            pltpu.VMEM((NCHUNK, CHUNK), jnp.int32),
            pltpu.VMEM((BPW, EMBED_DIM), jnp.float32),
            pltpu.VMEM((2, CHUNK, EMBED_DIM), jnp.float32),
            pltpu.VMEM((BPW,), jnp.float32),
            pltpu.SemaphoreType.DMA,
            pltpu.SemaphoreType.DMA,
        ],
    )
    def k(uemb, memb, ubt, mbt, uidx, midx, out_u, out_m, out_ub, out_mb,
          idx_v, ridx_v, rows_v, bbuf, bias_v, sem, bsem):
        wid = lax.axis_index("s") * NC + lax.axis_index("c")
        base = wid * BPW
        row0 = wid * NCHUNK
        lanes = lax.iota(jnp.int32, NLANE)

        def gather_one(idx_hbm, table, bias_tab, out_rows, out_bias):
            pltpu.sync_copy(idx_hbm.at[pl.ds(row0, NCHUNK)], idx_v)
            row_cps = [
                pltpu.async_copy(
                    table.at[idx_v.at[j]],
                    rows_v.at[pl.ds(j * CHUNK, CHUNK)], sem)
                for j in range(NCHUNK)
            ]
            for j in range(NCHUNK):
                for b in range(CHUNK // NLANE):
                    iv = idx_v[j, pl.ds(b * NLANE, NLANE)]
                    ridx_v[j, pl.ds(b * NLANE, NLANE)] = iv >> 7
            bias_cps = [None, None]
            for j in range(2):
                bias_cps[j] = pltpu.async_copy(
                    bias_tab.at[ridx_v.at[j]], bbuf.at[j], bsem)
            for j in range(NCHUNK):
                bias_cps[j % 2].wait()
                for b in range(CHUNK // NLANE):
                    iv = idx_v[j, pl.ds(b * NLANE, NLANE)]
                    k_vec = lanes + (b * NLANE)
                    lane_vec = iv & 127
                    vals = plsc.load_gather(bbuf.at[j % 2], [k_vec, lane_vec])
                    bias_v[pl.ds(j * CHUNK + b * NLANE, NLANE)] = vals
                if j + 2 < NCHUNK:
                    bias_cps[j % 2] = pltpu.async_copy(
                        bias_tab.at[ridx_v.at[j + 2]], bbuf.at[j % 2], bsem)
            for c in row_cps:
                c.wait()
            pltpu.sync_copy(rows_v, out_rows.at[pl.ds(base, BPW)])
            pltpu.sync_copy(bias_v, out_bias.at[pl.ds(base, BPW)])

        gather_one(uidx, uemb, ubt, out_u, out_ub)
        gather_one(midx, memb, mbt, out_m, out_mb)

    return k(user_emb, movie_emb, ubt2, mbt2, uidx2, midx2)


B_BLK = 2048


def _mlp_body(u_ref, m_ref, g_ref, w1u_ref, w1m_ref, w1g_ref, b1_ref,
              w2_ref, ub_ref, mb_ref, c_ref, o_ref):
    h = jnp.dot(u_ref[...], w1u_ref[...], preferred_element_type=jnp.float32)
    h = h + jnp.dot(m_ref[...], w1m_ref[...], preferred_element_type=jnp.float32)
    h = h + jnp.dot(g_ref[...], w1g_ref[...], preferred_element_type=jnp.float32)
    h = jnp.maximum(h + b1_ref[...], 0.0)
    s = jnp.dot(h, w2_ref[...], preferred_element_type=jnp.float32)
    o_ref[...] = s + ub_ref[...] + mb_ref[...] + c_ref[0]


def _mlp(u_rows, m_rows, genre, w1u, w1m, w1g, b1r, W2, u_bias, m_bias, c):
    grid = (BATCH // B_BLK,)
    return pl.pallas_call(
        _mlp_body,
        grid=grid,
        in_specs=[
            pl.BlockSpec((B_BLK, EMBED_DIM), lambda i: (i, 0)),
            pl.BlockSpec((B_BLK, EMBED_DIM), lambda i: (i, 0)),
            pl.BlockSpec((B_BLK, NUM_GENRES), lambda i: (i, 0)),
            pl.BlockSpec((EMBED_DIM, HIDDEN_DIM), lambda i: (0, 0)),
            pl.BlockSpec((EMBED_DIM, HIDDEN_DIM), lambda i: (0, 0)),
            pl.BlockSpec((NUM_GENRES, HIDDEN_DIM), lambda i: (0, 0)),
            pl.BlockSpec((1, HIDDEN_DIM), lambda i: (0, 0)),
            pl.BlockSpec((HIDDEN_DIM, 1), lambda i: (0, 0)),
            pl.BlockSpec((B_BLK, 1), lambda i: (i, 0)),
            pl.BlockSpec((B_BLK, 1), lambda i: (i, 0)),
            pl.BlockSpec(memory_space=pltpu.SMEM),
        ],
        out_specs=pl.BlockSpec((B_BLK, 1), lambda i: (i, 0)),
        out_shape=jax.ShapeDtypeStruct((BATCH, 1), jnp.float32),
    )(u_rows, m_rows, genre, w1u, w1m, w1g, b1r, W2, u_bias, m_bias, c)


def _pad_bias(tab):
    flat = tab[:, 0]
    padded = jnp.pad(flat, (0, 782 * 128 - flat.shape[0]))
    return padded.reshape(782, 128)


def kernel(user_indices, movie_indices, genre_features, user_emb, movie_emb,
           user_bias_tab, movie_bias_tab, global_bias, W1, b1, W2, b2):
    uidx2 = user_indices.astype(jnp.int32).reshape(BATCH // CHUNK, CHUNK)
    midx2 = movie_indices.astype(jnp.int32).reshape(BATCH // CHUNK, CHUNK)
    ubt2 = _pad_bias(user_bias_tab)
    mbt2 = _pad_bias(movie_bias_tab)
    u_rows, m_rows, u_bias, m_bias = _sc_gather(
        user_emb, movie_emb, ubt2, mbt2, uidx2, midx2)
    w1u = W1[:EMBED_DIM]
    w1m = W1[EMBED_DIM:2 * EMBED_DIM]
    w1g = W1[2 * EMBED_DIM:]
    b1r = b1.reshape(1, HIDDEN_DIM)
    c = b2 + global_bias
    out = _mlp(u_rows, m_rows, genre_features, w1u, w1m, w1g, b1r, W2,
               u_bias.reshape(BATCH, 1), m_bias.reshape(BATCH, 1), c)
    return out[:, 0]


# speedup vs baseline: 2.0869x; 1.0969x over previous
import dataclasses
import functools

import jax
import jax.numpy as jnp
from jax import lax
from jax.experimental import pallas as pl
from jax.experimental.pallas import tpu as pltpu
from jax.experimental.pallas import tpu_sc as plsc

BATCH = 16384
EMBED_DIM = 128
NUM_GENRES = 32
HIDDEN_DIM = 1024

NC = 2
NS = 16
NW = NC * NS
BPW = BATCH // NW
CHUNK = 128
NCHUNK = BPW // CHUNK
NLANE = 16


def _sc_gather(user_emb, movie_emb, ubt2, mbt2, uidx2, midx2):
    mesh = plsc.VectorSubcoreMesh(core_axis_name="c", subcore_axis_name="s")
    out_types = (
        jax.ShapeDtypeStruct((BATCH, 2 * EMBED_DIM), jnp.float32),
        jax.ShapeDtypeStruct((BATCH,), jnp.float32),
        jax.ShapeDtypeStruct((BATCH,), jnp.float32),
    )

    cp = pltpu.CompilerParams()
    if "needs_layout_passes" in pltpu.CompilerParams.__dataclass_fields__:
        cp = dataclasses.replace(cp, needs_layout_passes=False)

    @functools.partial(
        pl.kernel,
        mesh=mesh,
        out_type=out_types,
        compiler_params=cp,
        scratch_types=[
            pltpu.VMEM((NCHUNK, CHUNK), jnp.int32),
            pltpu.VMEM((NCHUNK, CHUNK), jnp.int32),
            pltpu.VMEM((BPW, EMBED_DIM), jnp.float32),
            pltpu.VMEM((2, CHUNK, EMBED_DIM), jnp.float32),
            pltpu.VMEM((BPW,), jnp.float32),
            pltpu.SemaphoreType.DMA,
            pltpu.SemaphoreType.DMA,
        ],
    )
    def k(uemb, memb, ubt, mbt, uidx, midx, out_um, out_ub, out_mb,
          idx_v, ridx_v, rows_v, bbuf, bias_v, sem, bsem):
        wid = lax.axis_index("s") * NC + lax.axis_index("c")
        base = wid * BPW
        row0 = wid * NCHUNK
        lanes = lax.iota(jnp.int32, NLANE)

        def gather_one(idx_hbm, table, bias_tab, col0, out_bias):
            pltpu.sync_copy(idx_hbm.at[pl.ds(row0, NCHUNK)], idx_v)
            row_cps = [
                pltpu.async_copy(
                    table.at[idx_v.at[j]],
                    rows_v.at[pl.ds(j * CHUNK, CHUNK)], sem)
                for j in range(NCHUNK)
            ]
            for j in range(NCHUNK):
                for b in range(CHUNK // NLANE):
                    iv = idx_v[j, pl.ds(b * NLANE, NLANE)]
                    ridx_v[j, pl.ds(b * NLANE, NLANE)] = iv >> 7
            bias_cps = [None, None]
            for j in range(2):
                bias_cps[j] = pltpu.async_copy(
                    bias_tab.at[ridx_v.at[j]], bbuf.at[j], bsem)
            for j in range(NCHUNK):
                bias_cps[j % 2].wait()
                for b in range(CHUNK // NLANE):
                    iv = idx_v[j, pl.ds(b * NLANE, NLANE)]
                    k_vec = lanes + (b * NLANE)
                    lane_vec = iv & 127
                    vals = plsc.load_gather(bbuf.at[j % 2], [k_vec, lane_vec])
                    bias_v[pl.ds(j * CHUNK + b * NLANE, NLANE)] = vals
                if j + 2 < NCHUNK:
                    bias_cps[j % 2] = pltpu.async_copy(
                        bias_tab.at[ridx_v.at[j + 2]], bbuf.at[j % 2], bsem)
            for c in row_cps:
                c.wait()
            pltpu.sync_copy(
                rows_v,
                out_um.at[pl.ds(base, BPW), pl.ds(col0, EMBED_DIM)])
            pltpu.sync_copy(bias_v, out_bias.at[pl.ds(base, BPW)])

        gather_one(uidx, uemb, ubt, 0, out_ub)
        gather_one(midx, memb, mbt, EMBED_DIM, out_mb)

    return k(user_emb, movie_emb, ubt2, mbt2, uidx2, midx2)


B_BLK = 2048


def _mlp_body(um_ref, g_ref, w1um_ref, w1g_ref, b1_ref,
              w2_ref, ub_ref, mb_ref, c_ref, o_ref):
    h = jnp.dot(um_ref[...], w1um_ref[...], preferred_element_type=jnp.float32)
    h = h + jnp.dot(g_ref[...], w1g_ref[...], preferred_element_type=jnp.float32)
    h = jnp.maximum(h + b1_ref[...], 0.0)
    s = jnp.dot(h, w2_ref[...], preferred_element_type=jnp.float32)
    o_ref[...] = s + ub_ref[...] + mb_ref[...] + c_ref[0]


def _mlp(um_rows, genre, w1um, w1g, b1r, W2, u_bias, m_bias, c):
    grid = (BATCH // B_BLK,)
    return pl.pallas_call(
        _mlp_body,
        grid=grid,
        in_specs=[
            pl.BlockSpec((B_BLK, 2 * EMBED_DIM), lambda i: (i, 0)),
            pl.BlockSpec((B_BLK, NUM_GENRES), lambda i: (i, 0)),
            pl.BlockSpec((2 * EMBED_DIM, HIDDEN_DIM), lambda i: (0, 0)),
            pl.BlockSpec((NUM_GENRES, HIDDEN_DIM), lambda i: (0, 0)),
            pl.BlockSpec((1, HIDDEN_DIM), lambda i: (0, 0)),
            pl.BlockSpec((HIDDEN_DIM, 1), lambda i: (0, 0)),
            pl.BlockSpec((B_BLK, 1), lambda i: (i, 0)),
            pl.BlockSpec((B_BLK, 1), lambda i: (i, 0)),
            pl.BlockSpec(memory_space=pltpu.SMEM),
        ],
        out_specs=pl.BlockSpec((B_BLK, 1), lambda i: (i, 0)),
        out_shape=jax.ShapeDtypeStruct((BATCH, 1), jnp.float32),
    )(um_rows, genre, w1um, w1g, b1r, W2, u_bias, m_bias, c)


def _pad_bias(tab):
    flat = tab[:, 0]
    padded = jnp.pad(flat, (0, 782 * 128 - flat.shape[0]))
    return padded.reshape(782, 128)


def kernel(user_indices, movie_indices, genre_features, user_emb, movie_emb,
           user_bias_tab, movie_bias_tab, global_bias, W1, b1, W2, b2):
    uidx2 = user_indices.astype(jnp.int32).reshape(BATCH // CHUNK, CHUNK)
    midx2 = movie_indices.astype(jnp.int32).reshape(BATCH // CHUNK, CHUNK)
    ubt2 = _pad_bias(user_bias_tab)
    mbt2 = _pad_bias(movie_bias_tab)
    um_rows, u_bias, m_bias = _sc_gather(
        user_emb, movie_emb, ubt2, mbt2, uidx2, midx2)
    w1um = W1[:2 * EMBED_DIM]
    w1g = W1[2 * EMBED_DIM:]
    b1r = b1.reshape(1, HIDDEN_DIM)
    c = b2 + global_bias
    out = _mlp(um_rows, genre_features, w1um, w1g, b1r, W2,
               u_bias.reshape(BATCH, 1), m_bias.reshape(BATCH, 1), c)
    return out[:, 0]


# speedup vs baseline: 2.0936x; 1.0032x over previous
import dataclasses
import functools

import jax
import jax.numpy as jnp
from jax import lax
from jax.experimental import pallas as pl
from jax.experimental.pallas import tpu as pltpu
from jax.experimental.pallas import tpu_sc as plsc

BATCH = 16384
EMBED_DIM = 128
NUM_GENRES = 32
HIDDEN_DIM = 1024

NC = 2
NS = 16
NW = NC * NS
BPW = BATCH // NW
CHUNK = 128
NCHUNK = BPW // CHUNK
NLANE = 16


def _sc_gather(user_emb, movie_emb, ubt2, mbt2, uidx2, midx2):
    mesh = plsc.VectorSubcoreMesh(core_axis_name="c", subcore_axis_name="s")
    out_types = (
        jax.ShapeDtypeStruct((BATCH, 2 * EMBED_DIM), jnp.float32),
        jax.ShapeDtypeStruct((BATCH,), jnp.float32),
        jax.ShapeDtypeStruct((BATCH,), jnp.float32),
    )

    cp = pltpu.CompilerParams()
    if "needs_layout_passes" in pltpu.CompilerParams.__dataclass_fields__:
        cp = dataclasses.replace(cp, needs_layout_passes=False)

    @functools.partial(
        pl.kernel,
        mesh=mesh,
        out_type=out_types,
        compiler_params=cp,
        scratch_types=[
            pltpu.VMEM((NCHUNK, CHUNK), jnp.int32),
            pltpu.VMEM((NCHUNK, CHUNK), jnp.int32),
            pltpu.VMEM((BPW, EMBED_DIM), jnp.float32),
            pltpu.VMEM((2, CHUNK, EMBED_DIM), jnp.float32),
            pltpu.VMEM((BPW,), jnp.float32),
            pltpu.SemaphoreType.DMA,
            pltpu.SemaphoreType.DMA,
        ],
    )
    def k(uemb, memb, ubt, mbt, uidx, midx, out_um, out_ub, out_mb,
          idx_v, ridx_v, rows_v, bbuf, bias_v, sem, bsem):
        wid = lax.axis_index("s") * NC + lax.axis_index("c")
        base = wid * BPW
        row0 = wid * NCHUNK
        lanes = lax.iota(jnp.int32, NLANE)

        def gather_one(idx_hbm, table, bias_tab, col0, out_bias):
            pltpu.sync_copy(idx_hbm.at[pl.ds(row0, NCHUNK)], idx_v)
            row_cps = [
                pltpu.async_copy(
                    table.at[idx_v.at[j]],
                    rows_v.at[pl.ds(j * CHUNK, CHUNK)], sem)
                for j in range(NCHUNK)
            ]
            for j in range(NCHUNK):
                for b in range(CHUNK // NLANE):
                    iv = idx_v[j, pl.ds(b * NLANE, NLANE)]
                    ridx_v[j, pl.ds(b * NLANE, NLANE)] = iv >> 7
            bias_cps = [None, None]
            for j in range(2):
                bias_cps[j] = pltpu.async_copy(
                    bias_tab.at[ridx_v.at[j]], bbuf.at[j], bsem)
            for j in range(NCHUNK):
                bias_cps[j % 2].wait()
                for b in range(CHUNK // NLANE):
                    iv = idx_v[j, pl.ds(b * NLANE, NLANE)]
                    k_vec = lanes + (b * NLANE)
                    lane_vec = iv & 127
                    vals = plsc.load_gather(bbuf.at[j % 2], [k_vec, lane_vec])
                    bias_v[pl.ds(j * CHUNK + b * NLANE, NLANE)] = vals
                if j + 2 < NCHUNK:
                    bias_cps[j % 2] = pltpu.async_copy(
                        bias_tab.at[ridx_v.at[j + 2]], bbuf.at[j % 2], bsem)
            for c in row_cps:
                c.wait()
            pltpu.sync_copy(
                rows_v,
                out_um.at[pl.ds(base, BPW), pl.ds(col0, EMBED_DIM)])
            pltpu.sync_copy(bias_v, out_bias.at[pl.ds(base, BPW)])

        gather_one(uidx, uemb, ubt, 0, out_ub)
        gather_one(midx, memb, mbt, EMBED_DIM, out_mb)

    return k(user_emb, movie_emb, ubt2, mbt2, uidx2, midx2)


B_BLK = 2048


def _mlp_body(um_ref, g_ref, w1um_ref, w1g_ref, b1_ref,
              w2_ref, ub_ref, mb_ref, c_ref, o_ref):
    um = um_ref[...].astype(jnp.bfloat16)
    g = g_ref[...].astype(jnp.bfloat16)
    h = jnp.dot(um, w1um_ref[...], preferred_element_type=jnp.float32)
    h = h + jnp.dot(g, w1g_ref[...], preferred_element_type=jnp.float32)
    h = jnp.maximum(h + b1_ref[...], 0.0).astype(jnp.bfloat16)
    s = jnp.dot(h, w2_ref[...], preferred_element_type=jnp.float32)
    o_ref[...] = s + ub_ref[...] + mb_ref[...] + c_ref[0]


def _mlp(um_rows, genre, w1um, w1g, b1r, W2, u_bias, m_bias, c):
    grid = (BATCH // B_BLK,)
    return pl.pallas_call(
        _mlp_body,
        grid=grid,
        in_specs=[
            pl.BlockSpec((B_BLK, 2 * EMBED_DIM), lambda i: (i, 0)),
            pl.BlockSpec((B_BLK, NUM_GENRES), lambda i: (i, 0)),
            pl.BlockSpec((2 * EMBED_DIM, HIDDEN_DIM), lambda i: (0, 0)),
            pl.BlockSpec((NUM_GENRES, HIDDEN_DIM), lambda i: (0, 0)),
            pl.BlockSpec((1, HIDDEN_DIM), lambda i: (0, 0)),
            pl.BlockSpec((HIDDEN_DIM, 1), lambda i: (0, 0)),
            pl.BlockSpec((B_BLK, 1), lambda i: (i, 0)),
            pl.BlockSpec((B_BLK, 1), lambda i: (i, 0)),
            pl.BlockSpec(memory_space=pltpu.SMEM),
        ],
        out_specs=pl.BlockSpec((B_BLK, 1), lambda i: (i, 0)),
        out_shape=jax.ShapeDtypeStruct((BATCH, 1), jnp.float32),
    )(um_rows, genre, w1um, w1g, b1r, W2, u_bias, m_bias, c)


def _pad_bias(tab):
    flat = tab[:, 0]
    padded = jnp.pad(flat, (0, 782 * 128 - flat.shape[0]))
    return padded.reshape(782, 128)


def kernel(user_indices, movie_indices, genre_features, user_emb, movie_emb,
           user_bias_tab, movie_bias_tab, global_bias, W1, b1, W2, b2):
    uidx2 = user_indices.astype(jnp.int32).reshape(BATCH // CHUNK, CHUNK)
    midx2 = movie_indices.astype(jnp.int32).reshape(BATCH // CHUNK, CHUNK)
    ubt2 = _pad_bias(user_bias_tab)
    mbt2 = _pad_bias(movie_bias_tab)
    um_rows, u_bias, m_bias = _sc_gather(
        user_emb, movie_emb, ubt2, mbt2, uidx2, midx2)
    w1um = W1[:2 * EMBED_DIM].astype(jnp.bfloat16)
    w1g = W1[2 * EMBED_DIM:].astype(jnp.bfloat16)
    b1r = b1.reshape(1, HIDDEN_DIM)
    c = b2 + global_bias
    out = _mlp(um_rows, genre_features, w1um, w1g, b1r, W2.astype(jnp.bfloat16),
               u_bias.reshape(BATCH, 1), m_bias.reshape(BATCH, 1), c)
    return out[:, 0]


# speedup vs baseline: 2.1305x; 1.0176x over previous
import dataclasses
import functools

import jax
import jax.numpy as jnp
from jax import lax
from jax.experimental import pallas as pl
from jax.experimental.pallas import tpu as pltpu
from jax.experimental.pallas import tpu_sc as plsc

BATCH = 16384
EMBED_DIM = 128
NUM_GENRES = 32
HIDDEN_DIM = 1024

NC = 2
NS = 16
NW = NC * NS
BPW = BATCH // NW
CHUNK = 128
NCHUNK = BPW // CHUNK
NLANE = 16


def _sc_gather(user_emb, movie_emb, ubt2, mbt2, uidx2, midx2):
    mesh = plsc.VectorSubcoreMesh(core_axis_name="c", subcore_axis_name="s")
    out_types = (
        jax.ShapeDtypeStruct((BATCH, 2 * EMBED_DIM), jnp.float32),
        jax.ShapeDtypeStruct((BATCH,), jnp.float32),
    )

    cp = pltpu.CompilerParams()
    if "needs_layout_passes" in pltpu.CompilerParams.__dataclass_fields__:
        cp = dataclasses.replace(cp, needs_layout_passes=False)

    @functools.partial(
        pl.kernel,
        mesh=mesh,
        out_type=out_types,
        compiler_params=cp,
        scratch_types=[
            pltpu.VMEM((NCHUNK, CHUNK), jnp.int32),
            pltpu.VMEM((NCHUNK, CHUNK), jnp.int32),
            pltpu.VMEM((BPW, EMBED_DIM), jnp.float32),
            pltpu.VMEM((2, CHUNK, EMBED_DIM), jnp.float32),
            pltpu.VMEM((BPW,), jnp.float32),
            pltpu.SemaphoreType.DMA,
            pltpu.SemaphoreType.DMA,
        ],
    )
    def k(uemb, memb, ubt, mbt, uidx, midx, out_um, out_bias,
          idx_v, ridx_v, rows_v, bbuf, bias_v, sem, bsem):
        wid = lax.axis_index("s") * NC + lax.axis_index("c")
        base = wid * BPW
        row0 = wid * NCHUNK
        lanes = lax.iota(jnp.int32, NLANE)

        def gather_one(idx_hbm, table, bias_tab, col0, accumulate):
            pltpu.sync_copy(idx_hbm.at[pl.ds(row0, NCHUNK)], idx_v)
            row_cps = [
                pltpu.async_copy(
                    table.at[idx_v.at[j]],
                    rows_v.at[pl.ds(j * CHUNK, CHUNK)], sem)
                for j in range(NCHUNK)
            ]
            for j in range(NCHUNK):
                for b in range(CHUNK // NLANE):
                    iv = idx_v[j, pl.ds(b * NLANE, NLANE)]
                    ridx_v[j, pl.ds(b * NLANE, NLANE)] = iv >> 7
            bias_cps = [None, None]
            for j in range(2):
                bias_cps[j] = pltpu.async_copy(
                    bias_tab.at[ridx_v.at[j]], bbuf.at[j], bsem)
            for j in range(NCHUNK):
                bias_cps[j % 2].wait()
                for b in range(CHUNK // NLANE):
                    iv = idx_v[j, pl.ds(b * NLANE, NLANE)]
                    k_vec = lanes + (b * NLANE)
                    lane_vec = iv & 127
                    vals = plsc.load_gather(bbuf.at[j % 2], [k_vec, lane_vec])
                    dst = pl.ds(j * CHUNK + b * NLANE, NLANE)
                    if accumulate:
                        bias_v[dst] = bias_v[dst] + vals
                    else:
                        bias_v[dst] = vals
                if j + 2 < NCHUNK:
                    bias_cps[j % 2] = pltpu.async_copy(
                        bias_tab.at[ridx_v.at[j + 2]], bbuf.at[j % 2], bsem)
            for c in row_cps:
                c.wait()
            pltpu.sync_copy(
                rows_v,
                out_um.at[pl.ds(base, BPW), pl.ds(col0, EMBED_DIM)])

        gather_one(uidx, uemb, ubt, 0, False)
        gather_one(midx, memb, mbt, EMBED_DIM, True)
        pltpu.sync_copy(bias_v, out_bias.at[pl.ds(base, BPW)])

    return k(user_emb, movie_emb, ubt2, mbt2, uidx2, midx2)


B_BLK = 2048


def _mlp_body(um_ref, g_ref, w1um_ref, w1g_ref, b1_ref, w2r_ref, o_ref):
    um = um_ref[...].astype(jnp.bfloat16)
    h = jnp.dot(um, w1um_ref[...], preferred_element_type=jnp.float32)
    h = h + jnp.dot(g_ref[...], w1g_ref[...], preferred_element_type=jnp.float32)
    h = jnp.maximum(h + b1_ref[...], 0.0)
    s = jax.lax.dot_general(
        h, w2r_ref[...], (((1,), (1,)), ((), ())),
        preferred_element_type=jnp.float32)
    o_ref[...] = s.reshape(B_BLK)


def _mlp(um_rows, genre, w1um, w1g, b1r, w2r):
    grid = (BATCH // B_BLK,)
    return pl.pallas_call(
        _mlp_body,
        grid=grid,
        in_specs=[
            pl.BlockSpec((B_BLK, 2 * EMBED_DIM), lambda i: (i, 0)),
            pl.BlockSpec((B_BLK, NUM_GENRES), lambda i: (i, 0)),
            pl.BlockSpec((2 * EMBED_DIM, HIDDEN_DIM), lambda i: (0, 0)),
            pl.BlockSpec((NUM_GENRES, HIDDEN_DIM), lambda i: (0, 0)),
            pl.BlockSpec((1, HIDDEN_DIM), lambda i: (0, 0)),
            pl.BlockSpec((1, HIDDEN_DIM), lambda i: (0, 0)),
        ],
        out_specs=pl.BlockSpec((B_BLK,), lambda i: (i,)),
        out_shape=jax.ShapeDtypeStruct((BATCH,), jnp.float32),
    )(um_rows, genre, w1um, w1g, b1r, w2r)


def _pad_bias(tab):
    flat = tab[:, 0]
    padded = jnp.pad(flat, (0, 782 * 128 - flat.shape[0]))
    return padded.reshape(782, 128)


def kernel(user_indices, movie_indices, genre_features, user_emb, movie_emb,
           user_bias_tab, movie_bias_tab, global_bias, W1, b1, W2, b2):
    uidx2 = user_indices.astype(jnp.int32).reshape(BATCH // CHUNK, CHUNK)
    midx2 = movie_indices.astype(jnp.int32).reshape(BATCH // CHUNK, CHUNK)
    ubt2 = _pad_bias(user_bias_tab)
    mbt2 = _pad_bias(movie_bias_tab)
    um_rows, bias_sum = _sc_gather(
        user_emb, movie_emb, ubt2, mbt2, uidx2, midx2)
    w1um = W1[:2 * EMBED_DIM].astype(jnp.bfloat16)
    w1g = W1[2 * EMBED_DIM:].astype(jnp.bfloat16)
    b1r = b1.reshape(1, HIDDEN_DIM)
    w2r = W2.reshape(1, HIDDEN_DIM)
    genre_bf = genre_features.astype(jnp.bfloat16)
    s = _mlp(um_rows, genre_bf, w1um, w1g, b1r, w2r)
    return s + bias_sum + (b2[0] + global_bias[0])


# speedup vs baseline: 2.3417x; 1.0991x over previous
import dataclasses
import functools

import jax
import jax.numpy as jnp
from jax import lax
from jax.experimental import pallas as pl
from jax.experimental.pallas import tpu as pltpu
from jax.experimental.pallas import tpu_sc as plsc

BATCH = 16384
EMBED_DIM = 128
NUM_GENRES = 32
HIDDEN_DIM = 1024

NC = 2
NS = 16
NW = NC * NS
BPW = BATCH // NW
CHUNK = 128
NCHUNK = BPW // CHUNK
NLANE = 16


def _sc_gather(user_emb, movie_emb, ubt2, mbt2, uidx2, midx2):
    mesh = plsc.VectorSubcoreMesh(core_axis_name="c", subcore_axis_name="s")
    out_types = (
        jax.ShapeDtypeStruct((BATCH, 2 * EMBED_DIM), jnp.float32),
        jax.ShapeDtypeStruct((BATCH,), jnp.float32),
    )

    cp = pltpu.CompilerParams()
    if "needs_layout_passes" in pltpu.CompilerParams.__dataclass_fields__:
        cp = dataclasses.replace(cp, needs_layout_passes=False)

    @functools.partial(
        pl.kernel,
        mesh=mesh,
        out_type=out_types,
        compiler_params=cp,
        scratch_types=[
            pltpu.VMEM((NCHUNK, CHUNK), jnp.int32),
            pltpu.VMEM((NCHUNK, CHUNK), jnp.int32),
            pltpu.VMEM((BPW, EMBED_DIM), jnp.float32),
            pltpu.VMEM((2, CHUNK, EMBED_DIM), jnp.float32),
            pltpu.VMEM((BPW,), jnp.float32),
            pltpu.SemaphoreType.DMA,
            pltpu.SemaphoreType.DMA,
        ],
    )
    def k(uemb, memb, ubt, mbt, uidx, midx, out_um, out_bias,
          idx_v, ridx_v, rows_v, bbuf, bias_v, sem, bsem):
        wid = lax.axis_index("s") * NC + lax.axis_index("c")
        base = wid * BPW
        row0 = wid * NCHUNK
        lanes = lax.iota(jnp.int32, NLANE)

        def gather_one(idx_hbm, table, bias_tab, col0, accumulate):
            pltpu.sync_copy(idx_hbm.at[pl.ds(row0, NCHUNK)], idx_v)
            row_cps = [
                pltpu.async_copy(
                    table.at[idx_v.at[j]],
                    rows_v.at[pl.ds(j * CHUNK, CHUNK)], sem)
                for j in range(NCHUNK)
            ]
            for j in range(NCHUNK):
                for b in range(CHUNK // NLANE):
                    iv = idx_v[j, pl.ds(b * NLANE, NLANE)]
                    ridx_v[j, pl.ds(b * NLANE, NLANE)] = iv >> 7
            bias_cps = [None, None]
            for j in range(2):
                bias_cps[j] = pltpu.async_copy(
                    bias_tab.at[ridx_v.at[j]], bbuf.at[j], bsem)
            for j in range(NCHUNK):
                bias_cps[j % 2].wait()
                for b in range(CHUNK // NLANE):
                    iv = idx_v[j, pl.ds(b * NLANE, NLANE)]
                    k_vec = lanes + (b * NLANE)
                    lane_vec = iv & 127
                    vals = plsc.load_gather(bbuf.at[j % 2], [k_vec, lane_vec])
                    dst = pl.ds(j * CHUNK + b * NLANE, NLANE)
                    if accumulate:
                        bias_v[dst] = bias_v[dst] + vals
                    else:
                        bias_v[dst] = vals
                if j + 2 < NCHUNK:
                    bias_cps[j % 2] = pltpu.async_copy(
                        bias_tab.at[ridx_v.at[j + 2]], bbuf.at[j % 2], bsem)
            for c in row_cps:
                c.wait()
            pltpu.sync_copy(
                rows_v,
                out_um.at[pl.ds(base, BPW), pl.ds(col0, EMBED_DIM)])

        gather_one(uidx, uemb, ubt, 0, False)
        gather_one(midx, memb, mbt, EMBED_DIM, True)
        pltpu.sync_copy(bias_v, out_bias.at[pl.ds(base, BPW)])

    return k(user_emb, movie_emb, ubt2, mbt2, uidx2, midx2)


B_BLK = 2048


def _mlp_body(um_ref, g_ref, w1_ref, b1_ref, w2_ref, o_ref):
    um = um_ref[...].astype(jnp.bfloat16)
    f = jnp.concatenate([um, g_ref[...]], axis=1)
    h = jnp.dot(f, w1_ref[...], preferred_element_type=jnp.float32)
    h = jnp.maximum(h + b1_ref[...], 0.0).astype(jnp.bfloat16)
    s = jnp.dot(h, w2_ref[...], preferred_element_type=jnp.float32)
    o_ref[...] = s.reshape(B_BLK)


def _mlp(um_rows, genre, w1, b1r, w2):
    grid = (BATCH // B_BLK,)
    return pl.pallas_call(
        _mlp_body,
        grid=grid,
        in_specs=[
            pl.BlockSpec((B_BLK, 2 * EMBED_DIM), lambda i: (i, 0)),
            pl.BlockSpec((B_BLK, NUM_GENRES), lambda i: (i, 0)),
            pl.BlockSpec((2 * EMBED_DIM + NUM_GENRES, HIDDEN_DIM),
                         lambda i: (0, 0)),
            pl.BlockSpec((1, HIDDEN_DIM), lambda i: (0, 0)),
            pl.BlockSpec((HIDDEN_DIM, 1), lambda i: (0, 0)),
        ],
        out_specs=pl.BlockSpec((B_BLK,), lambda i: (i,)),
        out_shape=jax.ShapeDtypeStruct((BATCH,), jnp.float32),
    )(um_rows, genre, w1, b1r, w2)


def _pad_bias(tab):
    flat = tab[:, 0]
    padded = jnp.pad(flat, (0, 782 * 128 - flat.shape[0]))
    return padded.reshape(782, 128)


def kernel(user_indices, movie_indices, genre_features, user_emb, movie_emb,
           user_bias_tab, movie_bias_tab, global_bias, W1, b1, W2, b2):
    uidx2 = user_indices.astype(jnp.int32).reshape(BATCH // CHUNK, CHUNK)
    midx2 = movie_indices.astype(jnp.int32).reshape(BATCH // CHUNK, CHUNK)
    ubt2 = _pad_bias(user_bias_tab)
    mbt2 = _pad_bias(movie_bias_tab)
    um_rows, bias_sum = _sc_gather(
        user_emb, movie_emb, ubt2, mbt2, uidx2, midx2)
    w1_bf = W1.astype(jnp.bfloat16)
    b1r = b1.reshape(1, HIDDEN_DIM)
    w2_bf = W2.astype(jnp.bfloat16)
    genre_bf = genre_features.astype(jnp.bfloat16)
    s = _mlp(um_rows, genre_bf, w1_bf, b1r, w2_bf)
    return s + bias_sum + (b2[0] + global_bias[0])


# speedup vs baseline: 2.5256x; 1.0785x over previous
import dataclasses
import functools

import jax
import jax.numpy as jnp
from jax import lax
from jax.experimental import pallas as pl
from jax.experimental.pallas import tpu as pltpu
from jax.experimental.pallas import tpu_sc as plsc

BATCH = 16384
EMBED_DIM = 128
NUM_GENRES = 32
HIDDEN_DIM = 1024

NC = 2
NS = 16
NW = NC * NS
BPW = BATCH // NW
CHUNK = 128
NCHUNK = BPW // CHUNK
NLANE = 16


def _sc_gather(user_emb, movie_emb, ubt2, mbt2, uidx2, midx2, nbatch):
    bpw = nbatch // NW
    nchunk = bpw // CHUNK
    mesh = plsc.VectorSubcoreMesh(core_axis_name="c", subcore_axis_name="s")
    out_types = (
        jax.ShapeDtypeStruct((nbatch, 2 * EMBED_DIM), jnp.float32),
        jax.ShapeDtypeStruct((nbatch,), jnp.float32),
    )

    cp = pltpu.CompilerParams()
    if "needs_layout_passes" in pltpu.CompilerParams.__dataclass_fields__:
        cp = dataclasses.replace(cp, needs_layout_passes=False)

    @functools.partial(
        pl.kernel,
        mesh=mesh,
        out_type=out_types,
        compiler_params=cp,
        scratch_types=[
            pltpu.VMEM((nchunk, CHUNK), jnp.int32),
            pltpu.VMEM((nchunk, CHUNK), jnp.int32),
            pltpu.VMEM((bpw, EMBED_DIM), jnp.float32),
            pltpu.VMEM((2, CHUNK, EMBED_DIM), jnp.float32),
            pltpu.VMEM((bpw,), jnp.float32),
            pltpu.SemaphoreType.DMA,
            pltpu.SemaphoreType.DMA,
        ],
    )
    def k(uemb, memb, ubt, mbt, uidx, midx, out_um, out_bias,
          idx_v, ridx_v, rows_v, bbuf, bias_v, sem, bsem):
        wid = lax.axis_index("s") * NC + lax.axis_index("c")
        base = wid * bpw
        row0 = wid * nchunk
        lanes = lax.iota(jnp.int32, NLANE)

        def gather_one(idx_hbm, table, bias_tab, col0, accumulate):
            pltpu.sync_copy(idx_hbm.at[pl.ds(row0, nchunk)], idx_v)
            row_cps = [
                pltpu.async_copy(
                    table.at[idx_v.at[j]],
                    rows_v.at[pl.ds(j * CHUNK, CHUNK)], sem)
                for j in range(nchunk)
            ]
            for j in range(nchunk):
                for b in range(CHUNK // NLANE):
                    iv = idx_v[j, pl.ds(b * NLANE, NLANE)]
                    ridx_v[j, pl.ds(b * NLANE, NLANE)] = iv >> 7
            nbuf = min(2, nchunk)
            bias_cps = [None] * nbuf
            for j in range(nbuf):
                bias_cps[j] = pltpu.async_copy(
                    bias_tab.at[ridx_v.at[j]], bbuf.at[j], bsem)
            for j in range(nchunk):
                bias_cps[j % nbuf].wait()
                for b in range(CHUNK // NLANE):
                    iv = idx_v[j, pl.ds(b * NLANE, NLANE)]
                    k_vec = lanes + (b * NLANE)
                    lane_vec = iv & 127
                    vals = plsc.load_gather(bbuf.at[j % nbuf], [k_vec, lane_vec])
                    dst = pl.ds(j * CHUNK + b * NLANE, NLANE)
                    if accumulate:
                        bias_v[dst] = bias_v[dst] + vals
                    else:
                        bias_v[dst] = vals
                if j + nbuf < nchunk:
                    bias_cps[j % nbuf] = pltpu.async_copy(
                        bias_tab.at[ridx_v.at[j + nbuf]], bbuf.at[j % nbuf],
                        bsem)
            for c in row_cps:
                c.wait()
            pltpu.sync_copy(
                rows_v,
                out_um.at[pl.ds(base, bpw), pl.ds(col0, EMBED_DIM)])

        gather_one(uidx, uemb, ubt, 0, False)
        gather_one(midx, memb, mbt, EMBED_DIM, True)
        pltpu.sync_copy(bias_v, out_bias.at[pl.ds(base, bpw)])

    return k(user_emb, movie_emb, ubt2, mbt2, uidx2, midx2)


B_BLK = 2048


def _mlp_body(um_ref, g_ref, w1_ref, b1_ref, w2_ref, o_ref):
    um = um_ref[...].astype(jnp.bfloat16)
    f = jnp.concatenate([um, g_ref[...]], axis=1)
    h = jnp.dot(f, w1_ref[...], preferred_element_type=jnp.float32)
    h = jnp.maximum(h + b1_ref[...], 0.0).astype(jnp.bfloat16)
    s = jnp.dot(h, w2_ref[...], preferred_element_type=jnp.float32)
    o_ref[...] = s.reshape(B_BLK)


def _mlp(um_rows, genre, w1, b1r, w2, nbatch):
    grid = (nbatch // B_BLK,)
    return pl.pallas_call(
        _mlp_body,
        grid=grid,
        in_specs=[
            pl.BlockSpec((B_BLK, 2 * EMBED_DIM), lambda i: (i, 0)),
            pl.BlockSpec((B_BLK, NUM_GENRES), lambda i: (i, 0)),
            pl.BlockSpec((2 * EMBED_DIM + NUM_GENRES, HIDDEN_DIM),
                         lambda i: (0, 0)),
            pl.BlockSpec((1, HIDDEN_DIM), lambda i: (0, 0)),
            pl.BlockSpec((HIDDEN_DIM, 1), lambda i: (0, 0)),
        ],
        out_specs=pl.BlockSpec((B_BLK,), lambda i: (i,)),
        out_shape=jax.ShapeDtypeStruct((nbatch,), jnp.float32),
    )(um_rows, genre, w1, b1r, w2)


def _pad_bias(tab):
    flat = tab[:, 0]
    padded = jnp.pad(flat, (0, 782 * 128 - flat.shape[0]))
    return padded.reshape(782, 128)


NSPLIT = 2
HBATCH = BATCH // NSPLIT


def kernel(user_indices, movie_indices, genre_features, user_emb, movie_emb,
           user_bias_tab, movie_bias_tab, global_bias, W1, b1, W2, b2):
    uidx = user_indices.astype(jnp.int32)
    midx = movie_indices.astype(jnp.int32)
    ubt2 = _pad_bias(user_bias_tab)
    mbt2 = _pad_bias(movie_bias_tab)
    w1_bf = W1.astype(jnp.bfloat16)
    b1r = b1.reshape(1, HIDDEN_DIM)
    w2_bf = W2.astype(jnp.bfloat16)
    genre_bf = genre_features.astype(jnp.bfloat16)

    gathered = []
    for k in range(NSPLIT):
        lo = k * HBATCH
        u2 = lax.dynamic_slice_in_dim(uidx, lo, HBATCH).reshape(
            HBATCH // CHUNK, CHUNK)
        m2 = lax.dynamic_slice_in_dim(midx, lo, HBATCH).reshape(
            HBATCH // CHUNK, CHUNK)
        gathered.append(
            _sc_gather(user_emb, movie_emb, ubt2, mbt2, u2, m2, HBATCH))
    outs = []
    for k in range(NSPLIT):
        um_rows, bias_sum = gathered[k]
        g = lax.dynamic_slice_in_dim(genre_bf, k * HBATCH, HBATCH)
        s = _mlp(um_rows, g, w1_bf, b1r, w2_bf, HBATCH)
        outs.append(s + bias_sum)
    return jnp.concatenate(outs) + (b2[0] + global_bias[0])
